# baseline (speedup 1.0000x reference)
import functools

import jax
import jax.numpy as jnp
from jax import lax
from jax.experimental import pallas as pl
from jax.experimental.pallas import tpu as pltpu
from jax.experimental.pallas import tpu_sc as plsc

N = 10000
NP = 10112
D = 128
DH = 64
E = 320000
NCLS = 40
NC = 2
NS = 16
NW = NC * NS
CHUNK = 128
E_PAD = 327680
E_TILE = E_PAD // NS
HOP_CHUNKS = E_TILE // CHUNK
N_STAGE = 8
STAGE_CHUNKS = HOP_CHUNKS // N_STAGE
STAGE_EDGES = STAGE_CHUNKS * CHUNK
RING = 2
STAGE_OUTER = STAGE_CHUNKS // RING
E_WORK = E_PAD // NW
W_CHUNKS = E_WORK // CHUNK
W_VECS = E_WORK // 16
ROWS_TILE = NP // NS

_mesh = functools.partial(
    plsc.VectorSubcoreMesh,
    core_axis_name="c", subcore_axis_name="s", num_cores=NC, num_subcores=NS,
)
_sc_params = pltpu.CompilerParams(
    needs_layout_passes=False, use_tc_tiling_on_sc=False
)


@functools.partial(
    pl.kernel,
    out_type=jax.ShapeDtypeStruct((NC, NP, 16), jnp.float32),
    mesh=_mesh(),
    scratch_types=[
        pltpu.VMEM((W_CHUNKS, CHUNK), jnp.int32),
        pltpu.VMEM((E_WORK,), jnp.float32),
        pltpu.VMEM((2, CHUNK, 16), jnp.float32),
        pltpu.VMEM((ROWS_TILE, 16), jnp.float32),
        pltpu.VMEM_SHARED((NP, 16), jnp.float32),
        [pltpu.SemaphoreType.DMA] * 2,
    ],
    compiler_params=_sc_params,
)
def _deg_kernel(col3, ew3, out, cbuf, ebuf, srows, zbuf, acc, dsems):
    c = lax.axis_index("c")
    s = lax.axis_index("s")
    w = c * NS + s

    def zero_row(i, _):
        zbuf[i, :] = jnp.zeros((16,), jnp.float32)
        return 0
    lax.fori_loop(0, ROWS_TILE, zero_row, 0)
    pltpu.sync_copy(zbuf, acc.at[pl.ds(s * ROWS_TILE, ROWS_TILE)])
    plsc.subcore_barrier()

    pltpu.sync_copy(col3.at[w], cbuf)
    pltpu.sync_copy(ew3.at[w], ebuf)

    def pair_body(g, _):
        for b in range(2):
            j = 2 * g + b

            @pl.when(g > 0)
            def _wait_prev():
                pltpu.make_async_copy(srows.at[b], acc.at[cbuf.at[j]],
                                      dsems[b]).wait()

            def edge_body(e, _):
                ew16 = plsc.load_gather(
                    ebuf, [jnp.full((16,), j * CHUNK + e, jnp.int32)])
                srows[b, e, :] = ew16
                return 0
            lax.fori_loop(0, CHUNK, edge_body, 0)
            pltpu.async_copy(srows.at[b], acc.at[cbuf.at[j]], dsems[b],
                             add=True)
        return 0
    lax.fori_loop(0, W_CHUNKS // 2, pair_body, 0)
    for b in range(2):
        pltpu.make_async_copy(srows.at[b], acc.at[cbuf.at[b]],
                              dsems[b]).wait()
    plsc.subcore_barrier()
    pltpu.sync_copy(acc.at[pl.ds(s * ROWS_TILE, ROWS_TILE)],
                    out.at[c, pl.ds(s * ROWS_TILE, ROWS_TILE)])


@functools.partial(
    pl.kernel,
    out_type=jax.ShapeDtypeStruct((NW, W_VECS, 16), jnp.float32),
    mesh=_mesh(),
    scratch_types=[
        pltpu.VMEM((NP,), jnp.float32),
        pltpu.VMEM((W_VECS, 16), jnp.int32),
        pltpu.VMEM((W_VECS, 16), jnp.int32),
        pltpu.VMEM((W_VECS, 16), jnp.float32),
        pltpu.VMEM((W_VECS, 16), jnp.float32),
    ],
    compiler_params=_sc_params,
)
def _norm_kernel(row3, col3, ew3, dis, out, disv, rbuf, cbuf, ebuf, nbuf):
    c = lax.axis_index("c")
    s = lax.axis_index("s")
    w = c * NS + s
    pltpu.sync_copy(dis, disv)
    pltpu.sync_copy(row3.at[w], rbuf)
    pltpu.sync_copy(col3.at[w], cbuf)
    pltpu.sync_copy(ew3.at[w], ebuf)

    def body(i, _):
        dr = plsc.load_gather(disv, [rbuf[i]])
        dc = plsc.load_gather(disv, [cbuf[i]])
        nbuf[i, :] = dr * ebuf[i] * dc
        return 0
    lax.fori_loop(0, W_VECS, body, 0)
    pltpu.sync_copy(nbuf, out.at[w])


@functools.partial(
    pl.kernel,
    out_type=jax.ShapeDtypeStruct((NP, D), jnp.float32),
    mesh=_mesh(),
    scratch_types=[
        pltpu.VMEM((2, STAGE_CHUNKS, CHUNK), jnp.int32),
        pltpu.VMEM((2, STAGE_CHUNKS * 2, 64), jnp.int32),
        pltpu.VMEM((2, STAGE_EDGES), jnp.float32),
        pltpu.VMEM((RING, CHUNK, DH), jnp.float32),
        pltpu.VMEM((RING, CHUNK, DH), jnp.float32),
        pltpu.VMEM((ROWS_TILE,), jnp.float32),
        pltpu.VMEM_SHARED((NP, DH), jnp.float32),
        pltpu.VMEM_SHARED((NP, DH), jnp.float32),
        [pltpu.SemaphoreType.DMA] * RING,
        [pltpu.SemaphoreType.DMA] * RING,
        [pltpu.SemaphoreType.DMA] * RING,
        pltpu.SemaphoreType.DMA,
    ],
    compiler_params=_sc_params,
)
def _layer_sc(xfull, sn, row3, col3, norm3, yout,
              rbuf, cbuf, nbuf, rg, rs, snb, spma, spmb, gsems, ssems, ssems2, tsem):
    c = lax.axis_index("c")
    s = lax.axis_index("s")
    rslice = pl.ds(s * ROWS_TILE, ROWS_TILE)
    dslice = pl.ds(c * DH, DH)

    def scale_rows(src_spm, dst_spm):
        for off, ln in ((0, 128), (128, 128), (256, 128), (384, 128),
                        (512, 120)):
            blk = pl.ds(s * ROWS_TILE + off, ln)
            pltpu.sync_copy(src_spm.at[blk], rg.at[0, pl.ds(0, ln)])

            def init_body(i, _):
                nsplat = plsc.load_gather(
                    snb, [jnp.full((16,), off + i, jnp.int32)])
                for k in range(DH // 16):
                    sl = pl.ds(k * 16, 16)
                    rs[0, i, sl] = rg[0, i, sl] * nsplat
                return 0
            lax.fori_loop(0, ln, init_body, 0)
            pltpu.sync_copy(rs.at[0, pl.ds(0, ln)], dst_spm.at[blk])

    pltpu.sync_copy(sn.at[rslice], snb)
    pltpu.sync_copy(xfull.at[rslice, dslice], spma.at[rslice])
    scale_rows(spma, spmb)
    plsc.subcore_barrier()

    def run_hop(src, dst):
        for stg in range(N_STAGE):
            p = stg % 2
            cb = stg * STAGE_CHUNKS
            if stg == 0:
                pltpu.sync_copy(row3.at[s, pl.ds(cb, STAGE_CHUNKS)],
                                rbuf.at[p])
                pltpu.sync_copy(col3.at[s, pl.ds(cb * 2, STAGE_CHUNKS * 2)],
                                cbuf.at[p])
                pltpu.sync_copy(norm3.at[s, pl.ds(cb * CHUNK, STAGE_EDGES)],
                                nbuf.at[p])
            else:
                pltpu.make_async_copy(row3.at[s, pl.ds(cb, STAGE_CHUNKS)],
                                      rbuf.at[p], tsem).wait()
                pltpu.make_async_copy(
                    norm3.at[s, pl.ds(cb * CHUNK, STAGE_EDGES)], nbuf.at[p],
                    tsem).wait()
                pltpu.make_async_copy(
                    col3.at[s, pl.ds(cb * 2, STAGE_CHUNKS * 2)],
                    cbuf.at[p], tsem).wait()
            for b in range(RING):
                pltpu.async_copy(src.at[rbuf.at[p, b]], rg.at[b], gsems[b])

            first_stage = (stg == 0)
            last_stage = (stg + 1 == N_STAGE)
            nxt = (stg + 1) * STAGE_CHUNKS

            def outer_body(g, _):
                j0 = g * RING
                for b in range(RING):
                    j = j0 + b
                    pltpu.make_async_copy(src.at[rbuf.at[p, j]], rg.at[b],
                                          gsems[b]).wait()

                    if first_stage:
                        @pl.when(g > 0)
                        def _wait_prev_scatter():
                            pltpu.make_async_copy(
                                rs.at[b, pl.ds(0, 64)],
                                dst.at[cbuf.at[p, 0]], ssems[b]).wait()
                            pltpu.make_async_copy(
                                rs.at[b, pl.ds(64, 64)],
                                dst.at[cbuf.at[p, 1]], ssems2[b]).wait()
                    else:
                        pltpu.make_async_copy(
                            rs.at[b, pl.ds(0, 64)],
                            dst.at[cbuf.at[p, 0]], ssems[b]).wait()
                        pltpu.make_async_copy(
                            rs.at[b, pl.ds(64, 64)],
                            dst.at[cbuf.at[p, 1]], ssems2[b]).wait()

                    if not last_stage:
                        if b == 0:
                            @pl.when(g == 1)
                            def _prefire_staging():
                                pltpu.async_copy(
                                    row3.at[s, pl.ds(nxt, STAGE_CHUNKS)],
                                    rbuf.at[1 - p], tsem)
                                pltpu.async_copy(
                                    norm3.at[s, pl.ds(nxt * CHUNK,
                                                      STAGE_EDGES)],
                                    nbuf.at[1 - p], tsem)
                                pltpu.async_copy(
                                    col3.at[s, pl.ds(nxt * 2,
                                                     STAGE_CHUNKS * 2)],
                                    cbuf.at[1 - p], tsem)

                    def scale_body(i, _):
                        for u in range(4):
                            e = i * 4 + u
                            nsplat = plsc.load_gather(
                                nbuf.at[p],
                                [jnp.full((16,), j * CHUNK + e, jnp.int32)])
                            for k in range(DH // 16):
                                sl = pl.ds(k * 16, 16)
                                rs[b, e, sl] = rg[b, e, sl] * nsplat
                        return 0
                    lax.fori_loop(0, CHUNK // 4, scale_body, 0)

                    @pl.when(g < STAGE_OUTER - 1)
                    def _next_gather():
                        pltpu.async_copy(src.at[rbuf.at[p, j + RING]],
                                         rg.at[b], gsems[b])

                    pltpu.async_copy(rs.at[b, pl.ds(0, 64)],
                                     dst.at[cbuf.at[p, 2 * j]],
                                     ssems[b], add=True)
                    pltpu.async_copy(rs.at[b, pl.ds(64, 64)],
                                     dst.at[cbuf.at[p, 2 * j + 1]],
                                     ssems2[b], add=True)
                return 0
            lax.fori_loop(0, STAGE_OUTER, outer_body, 0)

        for b in range(RING):
            pltpu.make_async_copy(rs.at[b, pl.ds(0, 64)],
                                  dst.at[cbuf.at[0, 0]], ssems[b]).wait()
            pltpu.make_async_copy(rs.at[b, pl.ds(64, 64)],
                                  dst.at[cbuf.at[0, 1]], ssems2[b]).wait()

    run_hop(spma, spmb)
    plsc.subcore_barrier()

    scale_rows(spmb, spma)
    plsc.subcore_barrier()

    run_hop(spmb, spma)
    plsc.subcore_barrier()
    pltpu.sync_copy(spma.at[rslice], yout.at[rslice, dslice])


def _prep_body(degp_ref, dis_ref, sn_ref):
    deg = degp_ref[0, :, 0] + degp_ref[1, :, 0] + 1.0
    dis_ref[...] = lax.rsqrt(deg)
    sn_ref[...] = 1.0 / deg


def _prep_call(degp):
    return pl.pallas_call(
        _prep_body,
        out_shape=[
            jax.ShapeDtypeStruct((NP,), jnp.float32),
            jax.ShapeDtypeStruct((NP,), jnp.float32),
        ],
    )(degp)


def _layer_body(y_ref, w_ref, b_ref, h_ref):
    h = lax.dot_general(y_ref[...], w_ref[...], (((1,), (1,)), ((), ())),
                        preferred_element_type=jnp.float32)
    h_ref[...] = jnp.maximum(h + b_ref[...][None, :], 0.0)


def _layer_call(y, w1, b1):
    return pl.pallas_call(
        _layer_body,
        out_shape=jax.ShapeDtypeStruct((NP, D), jnp.float32),
    )(y, w1, b1)


def _final_body(y_ref, w_ref, b_ref, out_ref):
    o = lax.dot_general(y_ref[...], w_ref[...], (((1,), (1,)), ((), ())),
                        preferred_element_type=jnp.float32)
    o = o + b_ref[...][None, :]
    m = jnp.max(o, axis=1, keepdims=True)
    z = o - m
    lse = jnp.log(jnp.sum(jnp.exp(z), axis=1, keepdims=True))
    out_ref[...] = (z - lse)[:N, :]


def _final_call(yflat, w2, b2):
    return pl.pallas_call(
        _final_body,
        out_shape=jax.ShapeDtypeStruct((N, NCLS), jnp.float32),
    )(yflat, w2, b2)


def kernel(x, edge_index, edge_attr, W1, b1, W2, b2):
    row = edge_index[0]
    col = edge_index[1]
    xp = jnp.pad(x, ((0, NP - N), (0, 0)))
    pad = E_PAD - E
    rowp = jnp.pad(row, (0, pad))
    colp = jnp.pad(col, (0, pad))
    ewp = jnp.pad(edge_attr, (0, pad))

    col_w = colp.reshape(NW, W_CHUNKS, CHUNK)
    ew_w = ewp.reshape(NW, E_WORK)
    row_wv = rowp.reshape(NW, W_VECS, 16)
    col_wv = colp.reshape(NW, W_VECS, 16)
    ew_wv = ewp.reshape(NW, W_VECS, 16)
    row_t = rowp.reshape(NS, HOP_CHUNKS, CHUNK)
    col_t = colp.reshape(NS, HOP_CHUNKS * 2, 64)

    degp = _deg_kernel(col_w, ew_w)
    dis, sn = _prep_call(degp)
    norm = _norm_kernel(row_wv, col_wv, ew_wv, dis)
    norm_t = norm.reshape(NS, E_TILE)

    y = _layer_sc(xp, sn, row_t, col_t, norm_t)
    h = _layer_call(y, W1, b1)
    y2 = _layer_sc(h, sn, row_t, col_t, norm_t)
    return _final_call(y2, W2, b2)


# speedup vs baseline: 1.0078x; 1.0078x over previous
import functools

import jax
import jax.numpy as jnp
from jax import lax
from jax.experimental import pallas as pl
from jax.experimental.pallas import tpu as pltpu
from jax.experimental.pallas import tpu_sc as plsc

N = 10000
NP = 10112
D = 128
DH = 64
E = 320000
NCLS = 40
NC = 2
NS = 16
NW = NC * NS
CHUNK = 128
E_PAD = 327680
E_TILE = E_PAD // NS
HOP_CHUNKS = E_TILE // CHUNK
N_STAGE = 8
STAGE_CHUNKS = HOP_CHUNKS // N_STAGE
STAGE_EDGES = STAGE_CHUNKS * CHUNK
RING = 2
STAGE_OUTER = STAGE_CHUNKS // RING
E_WORK = E_PAD // NW
W_CHUNKS = E_WORK // CHUNK
W_VECS = E_WORK // 16
ROWS_TILE = NP // NS

_mesh = functools.partial(
    plsc.VectorSubcoreMesh,
    core_axis_name="c", subcore_axis_name="s", num_cores=NC, num_subcores=NS,
)
_sc_params = pltpu.CompilerParams(
    needs_layout_passes=False, use_tc_tiling_on_sc=False
)


@functools.partial(
    pl.kernel,
    out_type=jax.ShapeDtypeStruct((NC, NP, 16), jnp.float32),
    mesh=_mesh(),
    scratch_types=[
        pltpu.VMEM((W_CHUNKS, CHUNK), jnp.int32),
        pltpu.VMEM((E_WORK,), jnp.float32),
        pltpu.VMEM((2, CHUNK, 16), jnp.float32),
        pltpu.VMEM((ROWS_TILE, 16), jnp.float32),
        pltpu.VMEM_SHARED((NP, 16), jnp.float32),
        [pltpu.SemaphoreType.DMA] * 2,
    ],
    compiler_params=_sc_params,
)
def _deg_kernel(col3, ew3, out, cbuf, ebuf, srows, zbuf, acc, dsems):
    c = lax.axis_index("c")
    s = lax.axis_index("s")
    w = c * NS + s

    def zero_row(i, _):
        zbuf[i, :] = jnp.zeros((16,), jnp.float32)
        return 0
    lax.fori_loop(0, ROWS_TILE, zero_row, 0)
    pltpu.sync_copy(zbuf, acc.at[pl.ds(s * ROWS_TILE, ROWS_TILE)])
    plsc.subcore_barrier()

    pltpu.sync_copy(col3.at[w], cbuf)
    pltpu.sync_copy(ew3.at[w], ebuf)

    def pair_body(g, _):
        for b in range(2):
            j = 2 * g + b

            @pl.when(g > 0)
            def _wait_prev():
                pltpu.make_async_copy(srows.at[b], acc.at[cbuf.at[j]],
                                      dsems[b]).wait()

            def edge_body(e, _):
                ew16 = plsc.load_gather(
                    ebuf, [jnp.full((16,), j * CHUNK + e, jnp.int32)])
                srows[b, e, :] = ew16
                return 0
            lax.fori_loop(0, CHUNK, edge_body, 0)
            pltpu.async_copy(srows.at[b], acc.at[cbuf.at[j]], dsems[b],
                             add=True)
        return 0
    lax.fori_loop(0, W_CHUNKS // 2, pair_body, 0)
    for b in range(2):
        pltpu.make_async_copy(srows.at[b], acc.at[cbuf.at[b]],
                              dsems[b]).wait()
    plsc.subcore_barrier()
    pltpu.sync_copy(acc.at[pl.ds(s * ROWS_TILE, ROWS_TILE)],
                    out.at[c, pl.ds(s * ROWS_TILE, ROWS_TILE)])


@functools.partial(
    pl.kernel,
    out_type=jax.ShapeDtypeStruct((NW, W_VECS, 16), jnp.float32),
    mesh=_mesh(),
    scratch_types=[
        pltpu.VMEM((NP,), jnp.float32),
        pltpu.VMEM((W_VECS, 16), jnp.int32),
        pltpu.VMEM((W_VECS, 16), jnp.int32),
        pltpu.VMEM((W_VECS, 16), jnp.float32),
        pltpu.VMEM((W_VECS, 16), jnp.float32),
    ],
    compiler_params=_sc_params,
)
def _norm_kernel(row3, col3, ew3, dis, out, disv, rbuf, cbuf, ebuf, nbuf):
    c = lax.axis_index("c")
    s = lax.axis_index("s")
    w = c * NS + s
    pltpu.sync_copy(dis, disv)
    pltpu.sync_copy(row3.at[w], rbuf)
    pltpu.sync_copy(col3.at[w], cbuf)
    pltpu.sync_copy(ew3.at[w], ebuf)

    def body(i, _):
        dr = plsc.load_gather(disv, [rbuf[i]])
        dc = plsc.load_gather(disv, [cbuf[i]])
        nbuf[i, :] = dr * ebuf[i] * dc
        return 0
    lax.fori_loop(0, W_VECS, body, 0)
    pltpu.sync_copy(nbuf, out.at[w])


@functools.partial(
    pl.kernel,
    out_type=jax.ShapeDtypeStruct((NP, D), jnp.float32),
    mesh=_mesh(),
    scratch_types=[
        pltpu.VMEM((2, STAGE_CHUNKS, CHUNK), jnp.int32),
        pltpu.VMEM((2, STAGE_CHUNKS, CHUNK), jnp.int32),
        pltpu.VMEM((2, STAGE_EDGES), jnp.float32),
        pltpu.VMEM((RING, CHUNK, DH), jnp.float32),
        pltpu.VMEM((RING, CHUNK, DH), jnp.float32),
        pltpu.VMEM((ROWS_TILE,), jnp.float32),
        pltpu.VMEM_SHARED((NP, DH), jnp.float32),
        pltpu.VMEM_SHARED((NP, DH), jnp.float32),
        [pltpu.SemaphoreType.DMA] * RING,
        [pltpu.SemaphoreType.DMA] * RING,
        pltpu.SemaphoreType.DMA,
    ],
    compiler_params=_sc_params,
)
def _layer_sc(xfull, sn, row3, col3, norm3, yout,
              rbuf, cbuf, nbuf, rg, rs, snb, spma, spmb, gsems, ssems, tsem):
    c = lax.axis_index("c")
    s = lax.axis_index("s")
    rslice = pl.ds(s * ROWS_TILE, ROWS_TILE)
    dslice = pl.ds(c * DH, DH)

    def scale_rows(src_spm, dst_spm):
        for off, ln in ((0, 128), (128, 128), (256, 128), (384, 128),
                        (512, 120)):
            blk = pl.ds(s * ROWS_TILE + off, ln)
            pltpu.sync_copy(src_spm.at[blk], rg.at[0, pl.ds(0, ln)])

            def init_body(i, _):
                nsplat = plsc.load_gather(
                    snb, [jnp.full((16,), off + i, jnp.int32)])
                for k in range(DH // 16):
                    sl = pl.ds(k * 16, 16)
                    rs[0, i, sl] = rg[0, i, sl] * nsplat
                return 0
            lax.fori_loop(0, ln, init_body, 0)
            pltpu.sync_copy(rs.at[0, pl.ds(0, ln)], dst_spm.at[blk])

    pltpu.sync_copy(sn.at[rslice], snb)
    pltpu.sync_copy(xfull.at[rslice, dslice], spma.at[rslice])
    scale_rows(spma, spmb)
    plsc.subcore_barrier()

    def run_hop(src, dst):
        for stg in range(N_STAGE):
            p = stg % 2
            cb = stg * STAGE_CHUNKS
            if stg == 0:
                pltpu.sync_copy(row3.at[s, pl.ds(cb, STAGE_CHUNKS)],
                                rbuf.at[p])
                pltpu.sync_copy(col3.at[s, pl.ds(cb, STAGE_CHUNKS)],
                                cbuf.at[p])
                pltpu.sync_copy(norm3.at[s, pl.ds(cb * CHUNK, STAGE_EDGES)],
                                nbuf.at[p])
            else:
                pltpu.make_async_copy(row3.at[s, pl.ds(cb, STAGE_CHUNKS)],
                                      rbuf.at[p], tsem).wait()
                pltpu.make_async_copy(
                    norm3.at[s, pl.ds(cb * CHUNK, STAGE_EDGES)], nbuf.at[p],
                    tsem).wait()
                pltpu.make_async_copy(col3.at[s, pl.ds(cb, STAGE_CHUNKS)],
                                      cbuf.at[p], tsem).wait()
            for b in range(RING):
                pltpu.async_copy(src.at[rbuf.at[p, b]], rg.at[b], gsems[b])

            first_stage = (stg == 0)
            last_stage = (stg + 1 == N_STAGE)
            nxt = (stg + 1) * STAGE_CHUNKS

            def outer_body(g, _):
                j0 = g * RING
                for b in range(RING):
                    j = j0 + b
                    pltpu.make_async_copy(src.at[rbuf.at[p, j]], rg.at[b],
                                          gsems[b]).wait()

                    if first_stage:
                        @pl.when(g > 0)
                        def _wait_prev_scatter():
                            pltpu.make_async_copy(rs.at[b],
                                                  dst.at[cbuf.at[p, 0]],
                                                  ssems[b]).wait()
                    else:
                        pltpu.make_async_copy(rs.at[b], dst.at[cbuf.at[p, 0]],
                                              ssems[b]).wait()

                    if not last_stage:
                        if b == 0:
                            @pl.when(g == 1)
                            def _prefire_staging():
                                pltpu.async_copy(
                                    row3.at[s, pl.ds(nxt, STAGE_CHUNKS)],
                                    rbuf.at[1 - p], tsem)
                                pltpu.async_copy(
                                    norm3.at[s, pl.ds(nxt * CHUNK,
                                                      STAGE_EDGES)],
                                    nbuf.at[1 - p], tsem)
                                pltpu.async_copy(
                                    col3.at[s, pl.ds(nxt, STAGE_CHUNKS)],
                                    cbuf.at[1 - p], tsem)

                    def scale_body(i, _):
                        for u in range(4):
                            e = i * 4 + u
                            nsplat = plsc.load_gather(
                                nbuf.at[p],
                                [jnp.full((16,), j * CHUNK + e, jnp.int32)])
                            for k in range(DH // 16):
                                sl = pl.ds(k * 16, 16)
                                rs[b, e, sl] = rg[b, e, sl] * nsplat
                        return 0
                    lax.fori_loop(0, CHUNK // 4, scale_body, 0)

                    @pl.when(g < STAGE_OUTER - 1)
                    def _next_gather():
                        pltpu.async_copy(src.at[rbuf.at[p, j + RING]],
                                         rg.at[b], gsems[b])

                    pltpu.async_copy(rs.at[b], dst.at[cbuf.at[p, j]],
                                     ssems[b], add=True)
                return 0
            lax.fori_loop(0, STAGE_OUTER, outer_body, 0)

        for b in range(RING):
            pltpu.make_async_copy(rs.at[b], dst.at[cbuf.at[0, 0]],
                                  ssems[b]).wait()

    run_hop(spma, spmb)
    plsc.subcore_barrier()

    scale_rows(spmb, spma)
    plsc.subcore_barrier()

    run_hop(spmb, spma)
    plsc.subcore_barrier()
    pltpu.sync_copy(spma.at[rslice], yout.at[rslice, dslice])


def _prep_body(degp_ref, dis_ref, sn_ref):
    deg = degp_ref[0, :, 0] + degp_ref[1, :, 0] + 1.0
    dis_ref[...] = lax.rsqrt(deg)
    sn_ref[...] = 1.0 / deg


def _prep_call(degp):
    return pl.pallas_call(
        _prep_body,
        out_shape=[
            jax.ShapeDtypeStruct((NP,), jnp.float32),
            jax.ShapeDtypeStruct((NP,), jnp.float32),
        ],
    )(degp)


def _layer_body(y_ref, w_ref, b_ref, h_ref):
    h = lax.dot_general(y_ref[...], w_ref[...], (((1,), (1,)), ((), ())),
                        preferred_element_type=jnp.float32)
    h_ref[...] = jnp.maximum(h + b_ref[...][None, :], 0.0)


def _layer_call(y, w1, b1):
    return pl.pallas_call(
        _layer_body,
        out_shape=jax.ShapeDtypeStruct((NP, D), jnp.float32),
    )(y, w1, b1)


def _final_body(y_ref, w_ref, b_ref, out_ref):
    o = lax.dot_general(y_ref[...], w_ref[...], (((1,), (1,)), ((), ())),
                        preferred_element_type=jnp.float32)
    o = o + b_ref[...][None, :]
    m = jnp.max(o, axis=1, keepdims=True)
    z = o - m
    lse = jnp.log(jnp.sum(jnp.exp(z), axis=1, keepdims=True))
    out_ref[...] = (z - lse)[:N, :]


def _final_call(yflat, w2, b2):
    return pl.pallas_call(
        _final_body,
        out_shape=jax.ShapeDtypeStruct((N, NCLS), jnp.float32),
    )(yflat, w2, b2)


def kernel(x, edge_index, edge_attr, W1, b1, W2, b2):
    row = edge_index[0]
    col = edge_index[1]
    xp = jnp.pad(x, ((0, NP - N), (0, 0)))
    pad = E_PAD - E
    rowp = jnp.pad(row, (0, pad))
    colp = jnp.pad(col, (0, pad))
    ewp = jnp.pad(edge_attr, (0, pad))

    col_w = colp.reshape(NW, W_CHUNKS, CHUNK)
    ew_w = ewp.reshape(NW, E_WORK)
    row_wv = rowp.reshape(NW, W_VECS, 16)
    col_wv = colp.reshape(NW, W_VECS, 16)
    ew_wv = ewp.reshape(NW, W_VECS, 16)
    row_t = rowp.reshape(NS, HOP_CHUNKS, CHUNK)
    col_t = colp.reshape(NS, HOP_CHUNKS, CHUNK)

    degp = _deg_kernel(col_w, ew_w)
    dis, sn = _prep_call(degp)
    norm = _norm_kernel(row_wv, col_wv, ew_wv, dis)
    norm_t = norm.reshape(NS, E_TILE)

    y = _layer_sc(xp, sn, row_t, col_t, norm_t)
    h = _layer_call(y, W1, b1)
    y2 = _layer_sc(h, sn, row_t, col_t, norm_t)
    return _final_call(y2, W2, b2)
